# baseline (speedup 1.0000x reference)
import jax
import jax.numpy as jnp
from jax.experimental import pallas as pl
from jax.experimental.pallas import tpu as pltpu

MAX_LEN = 50
BATCH = 1024
N_RULES = 256
N_TOKENS = 1000
COPY_LEN = 200


_K_RULE = 4
_K_TOKEN = 16
_K_COPY = 4
_N_DMAS = _K_RULE + _K_TOKEN + _K_COPY


def _dma_body(s_ref, r_in, t_in, c_in, r_out, t_out, c_out, sems):
    idx = 7
    copies = []
    q = 0
    for src, dst, k in (
        (r_in, r_out, _K_RULE),
        (t_in, t_out, _K_TOKEN),
        (c_in, c_out, _K_COPY),
    ):
        ch = BATCH // k
        for j in range(k):
            c = pltpu.make_async_copy(
                src.at[idx, pl.ds(j * ch, ch)],
                dst.at[pl.ds(j * ch, ch)],
                sems.at[q],
            )
            c.start()
            copies.append(c)
            q += 1
    for c in copies:
        c.wait()


def kernel(rule_prob, token_prob, copy_prob, length):
    grid_spec = pltpu.PrefetchScalarGridSpec(
        num_scalar_prefetch=1,
        grid=(1,),
        in_specs=[
            pl.BlockSpec(memory_space=pl.ANY),
            pl.BlockSpec(memory_space=pl.ANY),
            pl.BlockSpec(memory_space=pl.ANY),
        ],
        out_specs=[
            pl.BlockSpec(memory_space=pl.ANY),
            pl.BlockSpec(memory_space=pl.ANY),
            pl.BlockSpec(memory_space=pl.ANY),
        ],
        scratch_shapes=[pltpu.SemaphoreType.DMA((_N_DMAS,))],
    )
    out_shape = [
        jax.ShapeDtypeStruct((BATCH, N_RULES), jnp.float32),
        jax.ShapeDtypeStruct((BATCH, N_TOKENS), jnp.float32),
        jax.ShapeDtypeStruct((BATCH, COPY_LEN), jnp.float32),
    ]
    r, t, c = pl.pallas_call(
        _dma_body, grid_spec=grid_spec, out_shape=out_shape
    )(length, rule_prob, token_prob, copy_prob)
    return (r, t, c)


# speedup vs baseline: 2.3276x; 2.3276x over previous
import jax
import jax.numpy as jnp
from jax.experimental import pallas as pl
from jax.experimental.pallas import tpu as pltpu

MAX_LEN = 50
BATCH = 1024
N_RULES = 256
N_TOKENS = 1000
COPY_LEN = 200

_K_RULE = 4
_K_TOKEN = 5
_K_COPY = 5
_N_DMAS = _K_RULE + _K_TOKEN + _K_COPY


def _dma_body(s_ref, r_in, t_in, c_in, r_out, t_out, c_out, sems):
    idx = (s_ref[0] - 1) % MAX_LEN
    copies = []
    q = 0
    for src, dst, rows, k in (
        (r_in, r_out, BATCH, _K_RULE),
        (t_in, t_out, N_TOKENS, _K_TOKEN),
        (c_in, c_out, COPY_LEN, _K_COPY),
    ):
        ch = rows // k
        for j in range(k):
            c = pltpu.make_async_copy(
                src.at[idx, pl.ds(j * ch, ch)],
                dst.at[pl.ds(j * ch, ch)],
                sems.at[q],
            )
            c.start()
            copies.append(c)
            q += 1
    for c in copies:
        c.wait()


def kernel(rule_prob, token_prob, copy_prob, length):
    token_t = jnp.swapaxes(token_prob, 1, 2)
    copy_t = jnp.swapaxes(copy_prob, 1, 2)
    grid_spec = pltpu.PrefetchScalarGridSpec(
        num_scalar_prefetch=1,
        grid=(1,),
        in_specs=[
            pl.BlockSpec(memory_space=pl.ANY),
            pl.BlockSpec(memory_space=pl.ANY),
            pl.BlockSpec(memory_space=pl.ANY),
        ],
        out_specs=[
            pl.BlockSpec(memory_space=pl.ANY),
            pl.BlockSpec(memory_space=pl.ANY),
            pl.BlockSpec(memory_space=pl.ANY),
        ],
        scratch_shapes=[pltpu.SemaphoreType.DMA((_N_DMAS,))],
    )
    out_shape = [
        jax.ShapeDtypeStruct((BATCH, N_RULES), jnp.float32),
        jax.ShapeDtypeStruct((N_TOKENS, BATCH), jnp.float32),
        jax.ShapeDtypeStruct((COPY_LEN, BATCH), jnp.float32),
    ]
    r, t, c = pl.pallas_call(
        _dma_body, grid_spec=grid_spec, out_shape=out_shape
    )(length, rule_prob, token_t, copy_t)
    return (r, t.T, c.T)


# speedup vs baseline: 26.1625x; 11.2402x over previous
import jax
import jax.numpy as jnp
from jax.experimental import pallas as pl
from jax.experimental.pallas import tpu as pltpu

MAX_LEN = 50


def _copy_body(s_ref, in_ref, out_ref):
    del s_ref
    out_ref[...] = in_ref[0]


def _gather_slice(x, length, grid):
    _, rows, cols = x.shape
    br = rows // grid

    def im_in(i, s):
        idx = (s[0] - 1) % MAX_LEN
        return (idx, i, 0)

    def im_out(i, s):
        del s
        return (i, 0)

    grid_spec = pltpu.PrefetchScalarGridSpec(
        num_scalar_prefetch=1,
        grid=(grid,),
        in_specs=[pl.BlockSpec((1, br, cols), im_in)],
        out_specs=pl.BlockSpec((br, cols), im_out),
    )
    return pl.pallas_call(
        _copy_body,
        grid_spec=grid_spec,
        out_shape=jax.ShapeDtypeStruct((rows, cols), x.dtype),
    )(length, x)


def kernel(rule_prob, token_prob, copy_prob, length):
    token_t = jnp.swapaxes(token_prob, 1, 2)
    copy_t = jnp.swapaxes(copy_prob, 1, 2)
    r = _gather_slice(rule_prob, length, grid=8)
    t = _gather_slice(token_t, length, grid=5)
    c = _gather_slice(copy_t, length, grid=5)
    return (r, t.T, c.T)


# speedup vs baseline: 71.6288x; 2.7378x over previous
import jax
import jax.numpy as jnp
from jax.experimental import pallas as pl
from jax.experimental.pallas import tpu as pltpu

MAX_LEN = 50
BATCH = 1024
N_RULES = 256
N_TOKENS = 1000
COPY_LEN = 200

_PLANS = (
    (BATCH, N_RULES, 4),
    (N_TOKENS, BATCH, 5),
    (COPY_LEN, BATCH, 5),
)
_N_DMAS = sum(p[2] for p in _PLANS)


def _gather_body(s_ref, r_in, t_in, c_in, r_out, t_out, c_out,
                 r_buf, t_buf, c_buf, in_sems, out_sems):
    idx = (s_ref[0] - 1) % MAX_LEN

    ins = []
    outs = []
    q = 0
    for (src, dst, buf), (rows, _, k) in zip(
        ((r_in, r_out, r_buf), (t_in, t_out, t_buf), (c_in, c_out, c_buf)),
        _PLANS,
    ):
        ch = rows // k
        for j in range(k):
            sl = pl.ds(j * ch, ch)
            ins.append(
                pltpu.make_async_copy(src.at[idx, sl], buf.at[sl], in_sems.at[q])
            )
            outs.append(
                pltpu.make_async_copy(buf.at[sl], dst.at[sl], out_sems.at[q])
            )
            q += 1
    for c in ins:
        c.start()
    for cin, cout in zip(ins, outs):
        cin.wait()
        cout.start()
    for cout in outs:
        cout.wait()


def kernel(rule_prob, token_prob, copy_prob, length):
    token_t = jnp.swapaxes(token_prob, 1, 2)
    copy_t = jnp.swapaxes(copy_prob, 1, 2)

    grid_spec = pltpu.PrefetchScalarGridSpec(
        num_scalar_prefetch=1,
        grid=(1,),
        in_specs=[pl.BlockSpec(memory_space=pl.ANY)] * 3,
        out_specs=[pl.BlockSpec(memory_space=pl.ANY)] * 3,
        scratch_shapes=[
            pltpu.VMEM((BATCH, N_RULES), jnp.float32),
            pltpu.VMEM((N_TOKENS, BATCH), jnp.float32),
            pltpu.VMEM((COPY_LEN, BATCH), jnp.float32),
            pltpu.SemaphoreType.DMA((_N_DMAS,)),
            pltpu.SemaphoreType.DMA((_N_DMAS,)),
        ],
    )
    out_shape = [
        jax.ShapeDtypeStruct((BATCH, N_RULES), jnp.float32),
        jax.ShapeDtypeStruct((N_TOKENS, BATCH), jnp.float32),
        jax.ShapeDtypeStruct((COPY_LEN, BATCH), jnp.float32),
    ]
    r, t, c = pl.pallas_call(
        _gather_body, grid_spec=grid_spec, out_shape=out_shape
    )(length, rule_prob, token_t, copy_t)
    return (r, t.T, c.T)
